# baseline (speedup 1.0000x reference)
import functools
import math

import jax
import jax.numpy as jnp
from jax import lax
from jax.experimental import pallas as pl
from jax.experimental.pallas import tpu as pltpu
from jax.experimental.pallas import tpu_sc as plsc

N = 10000
E = 320000
D = 128
G = 64
C = 10

NP = 10240
NW = 32
CHUNK = 128
NCH = 80
NPH = 5
CPP = NCH // NPH
EP = NW * NCH * CHUNK
ROWS_PER_SUB = NP // 16
ZR = 16
NZB = ROWS_PER_SUB // ZR


def _sc_agg_body(h_hbm, srcs_hbm, dsts_hbm, out_hbm,
                 src_v, dst_v, rows0_v, rows1_v, zero_v, acc_sh,
                 sem0, sem1, semz):
    c = lax.axis_index("c")
    s = lax.axis_index("s")
    wid = s * 2 + c

    def zrow(i, carry):
        for l in range(8):
            zero_v[i, pl.ds(l * 16, 16)] = jnp.zeros((16,), jnp.float32)
        return carry
    lax.fori_loop(0, ZR, zrow, 0)

    def zblk(j, carry):
        pltpu.sync_copy(zero_v, acc_sh.at[pl.ds(s * ROWS_PER_SUB + j * ZR, ZR)])
        return carry
    lax.fori_loop(0, NZB, zblk, 0)
    plsc.subcore_barrier()

    pltpu.sync_copy(srcs_hbm.at[wid], src_v)
    pltpu.sync_copy(dsts_hbm.at[wid], dst_v)

    def body(j, carry):
        pltpu.async_copy(h_hbm.at[src_v.at[j]], rows0_v, sem0).wait()
        pltpu.sync_copy(rows0_v, acc_sh.at[dst_v.at[j]], add=True)
        return carry
    lax.fori_loop(0, NCH, body, 0)
    plsc.subcore_barrier()

    pltpu.sync_copy(acc_sh.at[pl.ds(s * ROWS_PER_SUB, ROWS_PER_SUB)],
                    out_hbm.at[c].at[pl.ds(s * ROWS_PER_SUB, ROWS_PER_SUB)])


@functools.cache
def _sc_agg_call():
    return pl.kernel(
        _sc_agg_body,
        out_type=jax.ShapeDtypeStruct((2, NP, D), jnp.float32),
        mesh=plsc.VectorSubcoreMesh(core_axis_name="c", subcore_axis_name="s"),
        scratch_types=[
            pltpu.VMEM((NCH, CHUNK), jnp.int32),
            pltpu.VMEM((NCH, CHUNK), jnp.int32),
            pltpu.VMEM((CHUNK, D), jnp.float32),
            pltpu.VMEM((CHUNK, D), jnp.float32),
            pltpu.VMEM((ZR, D), jnp.float32),
            pltpu.VMEM_SHARED((NP, D), jnp.float32),
            pltpu.SemaphoreType.DMA,
            pltpu.SemaphoreType.DMA,
            pltpu.SemaphoreType.DMA,
        ],
    )


def _sc_agg(h, src, dst):
    return _sc_agg_call()(h, src, dst)


BM = 512


def _bn_body(x_ref, g_ref, b_ref, o_ref):
    o_ref[...] = x_ref[...] * g_ref[...] + b_ref[...]


def _bn(x, gv, bv):
    return pl.pallas_call(
        _bn_body,
        grid=(NP // BM,),
        in_specs=[
            pl.BlockSpec((BM, D), lambda i: (i, 0)),
            pl.BlockSpec((1, D), lambda i: (0, 0)),
            pl.BlockSpec((1, D), lambda i: (0, 0)),
        ],
        out_specs=pl.BlockSpec((BM, D), lambda i: (i, 0)),
        out_shape=jax.ShapeDtypeStruct((NP, D), jnp.float32),
    )(x, gv, bv)


def _layer_body(a0_ref, a1_ref, h_ref, wr_ref, wo_ref, b_ref, o_ref, *, relu):
    agg = a0_ref[...] + a1_ref[...]
    acc = jnp.dot(agg, wr_ref[...], preferred_element_type=jnp.float32)
    acc = acc + jnp.dot(h_ref[...], wo_ref[...], preferred_element_type=jnp.float32)
    acc = acc + b_ref[...]
    if relu:
        acc = jnp.maximum(acc, 0.0)
    o_ref[...] = acc


def _layer(a0, a1, h, wr, wo, b, relu):
    return pl.pallas_call(
        functools.partial(_layer_body, relu=relu),
        grid=(NP // BM,),
        in_specs=[
            pl.BlockSpec((BM, D), lambda i: (i, 0)),
            pl.BlockSpec((BM, D), lambda i: (i, 0)),
            pl.BlockSpec((BM, D), lambda i: (i, 0)),
            pl.BlockSpec((D, D), lambda i: (0, 0)),
            pl.BlockSpec((D, D), lambda i: (0, 0)),
            pl.BlockSpec((1, D), lambda i: (0, 0)),
        ],
        out_specs=pl.BlockSpec((BM, D), lambda i: (i, 0)),
        out_shape=jax.ShapeDtypeStruct((NP, D), jnp.float32),
    )(a0, a1, h, wr, wo, b)


def _pool_body(h_ref, b_ref, wl_ref, bl_ref, o_ref, sums_ref, cnts_ref):
    i = pl.program_id(0)

    @pl.when(i == 0)
    def _():
        sums_ref[...] = jnp.zeros_like(sums_ref)
        cnts_ref[...] = jnp.zeros_like(cnts_ref)

    seg = b_ref[...]
    onehot = jnp.where(
        seg[:, None] == lax.broadcasted_iota(jnp.int32, (1, G), 1),
        1.0, 0.0).astype(jnp.float32)
    sums_ref[...] += lax.dot_general(
        onehot, h_ref[...], (((0,), (0,)), ((), ())),
        preferred_element_type=jnp.float32)
    cnts_ref[...] += jnp.sum(onehot, axis=0)[:, None]

    @pl.when(i == pl.num_programs(0) - 1)
    def _():
        pooled = sums_ref[...] / jnp.maximum(cnts_ref[...], 1.0)
        o_ref[...] = jnp.dot(pooled, wl_ref[...],
                             preferred_element_type=jnp.float32) + bl_ref[...]


def _pool(h, segs, wl, bl):
    return pl.pallas_call(
        _pool_body,
        grid=(NP // BM,),
        in_specs=[
            pl.BlockSpec((BM, D), lambda i: (i, 0)),
            pl.BlockSpec((BM,), lambda i: (i,)),
            pl.BlockSpec((D, D), lambda i: (0, 0)),
            pl.BlockSpec((1, D), lambda i: (0, 0)),
        ],
        out_specs=pl.BlockSpec((G, D), lambda i: (0, 0)),
        out_shape=jax.ShapeDtypeStruct((G, D), jnp.float32),
        scratch_shapes=[
            pltpu.VMEM((G, D), jnp.float32),
            pltpu.VMEM((G, D), jnp.float32),
        ],
    )(h, segs, wl, bl)


def kernel(x, edge_index, batch, bn_gamma, bn_beta,
           W1_rel, W1_root, b1, W2_rel, W2_root, b2,
           W3_rel, W3_root, b3, W_lin, b_lin):
    eps = 1e-5
    gv = (bn_gamma * (1.0 / math.sqrt(1.0 + eps)))[None, :]
    bv = bn_beta[None, :]

    xp = jnp.pad(x, ((0, NP - N), (0, 0)))
    src = jnp.pad(edge_index[0], (0, EP - E)).reshape(NW, NCH, CHUNK)
    dst = jnp.pad(edge_index[1], (0, EP - E), constant_values=N).reshape(NW, NCH, CHUNK)
    segs = jnp.pad(batch, (0, NP - N), constant_values=G).astype(jnp.int32)
    wl = jnp.pad(W_lin, ((0, 0), (0, D - C)))
    bl = jnp.pad(b_lin, (0, D - C))[None, :]

    h = _bn(xp, gv, bv)

    for (wr, wo, b, relu) in (
        (W1_rel, W1_root, b1, True),
        (W2_rel, W2_root, b2, True),
        (W3_rel, W3_root, b3, False),
    ):
        agg = _sc_agg(h, src, dst)
        h = _layer(agg[0], agg[1], h, wr, wo, b[None, :], relu)

    out = _pool(h, segs, wl, bl)
    return out[:, :C]


# speedup vs baseline: 1.5161x; 1.5161x over previous
import functools
import math

import jax
import jax.numpy as jnp
from jax import lax
from jax.experimental import pallas as pl
from jax.experimental.pallas import tpu as pltpu
from jax.experimental.pallas import tpu_sc as plsc

N = 10000
E = 320000
D = 128
G = 64
C = 10

NP = 10240
NW = 32
CHUNK = 128
NCH = 79
EP = NW * NCH * CHUNK
ROWS_PER_SUB = NP // 16
ZR = 16
NZB = ROWS_PER_SUB // ZR


def _sc_agg_body(h_hbm, srcs_hbm, dsts_hbm, out_hbm,
                 src_v, dst_v, rows0_v, zero_v, acc_sh, sem0):
    c = lax.axis_index("c")
    s = lax.axis_index("s")
    wid = s * 2 + c

    def zrow(i, carry):
        for l in range(8):
            zero_v[i, pl.ds(l * 16, 16)] = jnp.zeros((16,), jnp.float32)
        return carry
    lax.fori_loop(0, ZR, zrow, 0)

    def zblk(j, carry):
        pltpu.sync_copy(zero_v, acc_sh.at[pl.ds(s * ROWS_PER_SUB + j * ZR, ZR)])
        return carry
    lax.fori_loop(0, NZB, zblk, 0)
    plsc.subcore_barrier()

    pltpu.sync_copy(srcs_hbm.at[wid], src_v)
    pltpu.sync_copy(dsts_hbm.at[wid], dst_v)

    def body(j, carry):
        pltpu.async_copy(h_hbm.at[src_v.at[j]], rows0_v, sem0).wait()
        pltpu.sync_copy(rows0_v, acc_sh.at[dst_v.at[j]], add=True)
        return carry
    lax.fori_loop(0, NCH, body, 0)
    plsc.subcore_barrier()

    pltpu.sync_copy(acc_sh.at[pl.ds(s * ROWS_PER_SUB, ROWS_PER_SUB)],
                    out_hbm.at[c].at[pl.ds(s * ROWS_PER_SUB, ROWS_PER_SUB)])


@functools.cache
def _sc_agg_call():
    return pl.kernel(
        _sc_agg_body,
        out_type=jax.ShapeDtypeStruct((2, NP, D), jnp.float32),
        mesh=plsc.VectorSubcoreMesh(core_axis_name="c", subcore_axis_name="s"),
        scratch_types=[
            pltpu.VMEM((NCH, CHUNK), jnp.int32),
            pltpu.VMEM((NCH, CHUNK), jnp.int32),
            pltpu.VMEM((CHUNK, D), jnp.float32),
            pltpu.VMEM((ZR, D), jnp.float32),
            pltpu.VMEM_SHARED((NP, D), jnp.float32),
            pltpu.SemaphoreType.DMA,
        ],
    )


def _sc_agg(h, src, dst):
    return _sc_agg_call()(h, src, dst)


BM = 512


def _bn_body(x_ref, g_ref, b_ref, o_ref):
    o_ref[...] = x_ref[...] * g_ref[...] + b_ref[...]


def _bn(x, gv, bv):
    return pl.pallas_call(
        _bn_body,
        grid=(NP // BM,),
        in_specs=[
            pl.BlockSpec((BM, D), lambda i: (i, 0)),
            pl.BlockSpec((1, D), lambda i: (0, 0)),
            pl.BlockSpec((1, D), lambda i: (0, 0)),
        ],
        out_specs=pl.BlockSpec((BM, D), lambda i: (i, 0)),
        out_shape=jax.ShapeDtypeStruct((NP, D), jnp.float32),
    )(x, gv, bv)


def _layer_body(a0_ref, a1_ref, h_ref, wr_ref, wo_ref, b_ref, o_ref, *, relu):
    agg = a0_ref[...] + a1_ref[...]
    acc = jnp.dot(agg, wr_ref[...], preferred_element_type=jnp.float32)
    acc = acc + jnp.dot(h_ref[...], wo_ref[...], preferred_element_type=jnp.float32)
    acc = acc + b_ref[...]
    if relu:
        acc = jnp.maximum(acc, 0.0)
    o_ref[...] = acc


def _layer(a0, a1, h, wr, wo, b, relu):
    return pl.pallas_call(
        functools.partial(_layer_body, relu=relu),
        grid=(NP // BM,),
        in_specs=[
            pl.BlockSpec((BM, D), lambda i: (i, 0)),
            pl.BlockSpec((BM, D), lambda i: (i, 0)),
            pl.BlockSpec((BM, D), lambda i: (i, 0)),
            pl.BlockSpec((D, D), lambda i: (0, 0)),
            pl.BlockSpec((D, D), lambda i: (0, 0)),
            pl.BlockSpec((1, D), lambda i: (0, 0)),
        ],
        out_specs=pl.BlockSpec((BM, D), lambda i: (i, 0)),
        out_shape=jax.ShapeDtypeStruct((NP, D), jnp.float32),
    )(a0, a1, h, wr, wo, b)


def _pool_body(h_ref, b_ref, wl_ref, bl_ref, o_ref, sums_ref, cnts_ref):
    i = pl.program_id(0)

    @pl.when(i == 0)
    def _():
        sums_ref[...] = jnp.zeros_like(sums_ref)
        cnts_ref[...] = jnp.zeros_like(cnts_ref)

    seg = b_ref[...]
    onehot = jnp.where(
        seg[:, None] == lax.broadcasted_iota(jnp.int32, (1, G), 1),
        1.0, 0.0).astype(jnp.float32)
    sums_ref[...] += lax.dot_general(
        onehot, h_ref[...], (((0,), (0,)), ((), ())),
        preferred_element_type=jnp.float32)
    cnts_ref[...] += jnp.sum(onehot, axis=0)[:, None]

    @pl.when(i == pl.num_programs(0) - 1)
    def _():
        pooled = sums_ref[...] / jnp.maximum(cnts_ref[...], 1.0)
        o_ref[...] = jnp.dot(pooled, wl_ref[...],
                             preferred_element_type=jnp.float32) + bl_ref[...]


def _pool(h, segs, wl, bl):
    return pl.pallas_call(
        _pool_body,
        grid=(NP // BM,),
        in_specs=[
            pl.BlockSpec((BM, D), lambda i: (i, 0)),
            pl.BlockSpec((BM,), lambda i: (i,)),
            pl.BlockSpec((D, D), lambda i: (0, 0)),
            pl.BlockSpec((1, D), lambda i: (0, 0)),
        ],
        out_specs=pl.BlockSpec((G, D), lambda i: (0, 0)),
        out_shape=jax.ShapeDtypeStruct((G, D), jnp.float32),
        scratch_shapes=[
            pltpu.VMEM((G, D), jnp.float32),
            pltpu.VMEM((G, D), jnp.float32),
        ],
    )(h, segs, wl, bl)


def kernel(x, edge_index, batch, bn_gamma, bn_beta,
           W1_rel, W1_root, b1, W2_rel, W2_root, b2,
           W3_rel, W3_root, b3, W_lin, b_lin):
    eps = 1e-5
    gv = (bn_gamma * (1.0 / math.sqrt(1.0 + eps)))[None, :]
    bv = bn_beta[None, :]

    xp = jnp.pad(x, ((0, NP - N), (0, 0)))
    src = jnp.pad(edge_index[0], (0, EP - E)).reshape(NW, NCH, CHUNK)
    dst = jnp.pad(edge_index[1], (0, EP - E), constant_values=N).reshape(NW, NCH, CHUNK)
    segs = jnp.pad(batch, (0, NP - N), constant_values=G).astype(jnp.int32)
    wl = jnp.pad(W_lin, ((0, 0), (0, D - C)))
    bl = jnp.pad(b_lin, (0, D - C))[None, :]

    h = _bn(xp, gv, bv)

    for (wr, wo, b, relu) in (
        (W1_rel, W1_root, b1, True),
        (W2_rel, W2_root, b2, True),
        (W3_rel, W3_root, b3, False),
    ):
        agg = _sc_agg(h, src, dst)
        h = _layer(agg[0], agg[1], h, wr, wo, b[None, :], relu)

    out = _pool(h, segs, wl, bl)
    return out[:, :C]


# speedup vs baseline: 1.5167x; 1.0004x over previous
import functools
import math

import jax
import jax.numpy as jnp
from jax import lax
from jax.experimental import pallas as pl
from jax.experimental.pallas import tpu as pltpu
from jax.experimental.pallas import tpu_sc as plsc

N = 10000
E = 320000
D = 128
G = 64
C = 10

NP = 10240
NW = 32
CHUNK = 128
NCH = 79
EP = NW * NCH * CHUNK
ROWS_PER_SUB = NP // 16
ZR = 16
NZB = ROWS_PER_SUB // ZR


def _sc_agg_body(h_hbm, srcs_hbm, dsts_hbm, out_hbm,
                 src_v, dst_v, rows0_v, zero_v, acc_sh, sem0, sem1, semz):
    c = lax.axis_index("c")
    s = lax.axis_index("s")
    wid = s * 2 + c

    def zrow(i, carry):
        for l in range(8):
            zero_v[i, pl.ds(l * 16, 16)] = jnp.zeros((16,), jnp.float32)
        return carry
    lax.fori_loop(0, ZR, zrow, 0)

    def zblk(j, carry):
        pltpu.sync_copy(zero_v, acc_sh.at[pl.ds(s * ROWS_PER_SUB + j * ZR, ZR)])
        return carry
    lax.fori_loop(0, NZB, zblk, 0)
    plsc.subcore_barrier()

    pltpu.sync_copy(srcs_hbm.at[wid], src_v)
    pltpu.sync_copy(dsts_hbm.at[wid], dst_v)

    def body(j, carry):
        pltpu.async_copy(h_hbm.at[src_v.at[j]], rows0_v, sem0).wait()
        pltpu.sync_copy(rows0_v, acc_sh.at[dst_v.at[j]], add=True)
        return carry
    lax.fori_loop(0, NCH, body, 0)
    plsc.subcore_barrier()

    pltpu.sync_copy(acc_sh.at[pl.ds(s * ROWS_PER_SUB, ROWS_PER_SUB)],
                    out_hbm.at[c].at[pl.ds(s * ROWS_PER_SUB, ROWS_PER_SUB)])


@functools.cache
def _sc_agg_call():
    return pl.kernel(
        _sc_agg_body,
        out_type=jax.ShapeDtypeStruct((2, NP, D), jnp.float32),
        mesh=plsc.VectorSubcoreMesh(core_axis_name="c", subcore_axis_name="s"),
        scratch_types=[
            pltpu.VMEM((NCH, CHUNK), jnp.int32),
            pltpu.VMEM((NCH, CHUNK), jnp.int32),
            pltpu.VMEM((CHUNK, D), jnp.float32),
            pltpu.VMEM((ZR, D), jnp.float32),
            pltpu.VMEM_SHARED((NP, D), jnp.float32),
            pltpu.SemaphoreType.DMA,
            pltpu.SemaphoreType.DMA,
            pltpu.SemaphoreType.DMA,
        ],
    )


def _sc_agg(h, src, dst):
    return _sc_agg_call()(h, src, dst)


BM = 512


def _bn_body(x_ref, g_ref, b_ref, o_ref):
    o_ref[...] = x_ref[...] * g_ref[...] + b_ref[...]


def _bn(x, gv, bv):
    return pl.pallas_call(
        _bn_body,
        grid=(NP // BM,),
        in_specs=[
            pl.BlockSpec((BM, D), lambda i: (i, 0)),
            pl.BlockSpec((1, D), lambda i: (0, 0)),
            pl.BlockSpec((1, D), lambda i: (0, 0)),
        ],
        out_specs=pl.BlockSpec((BM, D), lambda i: (i, 0)),
        out_shape=jax.ShapeDtypeStruct((NP, D), jnp.float32),
    )(x, gv, bv)


def _layer_body(a0_ref, a1_ref, h_ref, wr_ref, wo_ref, b_ref, o_ref, *, relu):
    agg = a0_ref[...] + a1_ref[...]
    acc = jnp.dot(agg, wr_ref[...], preferred_element_type=jnp.float32)
    acc = acc + jnp.dot(h_ref[...], wo_ref[...], preferred_element_type=jnp.float32)
    acc = acc + b_ref[...]
    if relu:
        acc = jnp.maximum(acc, 0.0)
    o_ref[...] = acc


def _layer(a0, a1, h, wr, wo, b, relu):
    return pl.pallas_call(
        functools.partial(_layer_body, relu=relu),
        grid=(NP // BM,),
        in_specs=[
            pl.BlockSpec((BM, D), lambda i: (i, 0)),
            pl.BlockSpec((BM, D), lambda i: (i, 0)),
            pl.BlockSpec((BM, D), lambda i: (i, 0)),
            pl.BlockSpec((D, D), lambda i: (0, 0)),
            pl.BlockSpec((D, D), lambda i: (0, 0)),
            pl.BlockSpec((1, D), lambda i: (0, 0)),
        ],
        out_specs=pl.BlockSpec((BM, D), lambda i: (i, 0)),
        out_shape=jax.ShapeDtypeStruct((NP, D), jnp.float32),
    )(a0, a1, h, wr, wo, b)


def _pool_body(h_ref, b_ref, wl_ref, bl_ref, o_ref, sums_ref, cnts_ref):
    i = pl.program_id(0)

    @pl.when(i == 0)
    def _():
        sums_ref[...] = jnp.zeros_like(sums_ref)
        cnts_ref[...] = jnp.zeros_like(cnts_ref)

    seg = b_ref[...]
    onehot = jnp.where(
        seg[:, None] == lax.broadcasted_iota(jnp.int32, (1, G), 1),
        1.0, 0.0).astype(jnp.float32)
    sums_ref[...] += lax.dot_general(
        onehot, h_ref[...], (((0,), (0,)), ((), ())),
        preferred_element_type=jnp.float32)
    cnts_ref[...] += jnp.sum(onehot, axis=0)[:, None]

    @pl.when(i == pl.num_programs(0) - 1)
    def _():
        pooled = sums_ref[...] / jnp.maximum(cnts_ref[...], 1.0)
        o_ref[...] = jnp.dot(pooled, wl_ref[...],
                             preferred_element_type=jnp.float32) + bl_ref[...]


def _pool(h, segs, wl, bl):
    return pl.pallas_call(
        _pool_body,
        grid=(NP // BM,),
        in_specs=[
            pl.BlockSpec((BM, D), lambda i: (i, 0)),
            pl.BlockSpec((BM,), lambda i: (i,)),
            pl.BlockSpec((D, D), lambda i: (0, 0)),
            pl.BlockSpec((1, D), lambda i: (0, 0)),
        ],
        out_specs=pl.BlockSpec((G, D), lambda i: (0, 0)),
        out_shape=jax.ShapeDtypeStruct((G, D), jnp.float32),
        scratch_shapes=[
            pltpu.VMEM((G, D), jnp.float32),
            pltpu.VMEM((G, D), jnp.float32),
        ],
    )(h, segs, wl, bl)


def kernel(x, edge_index, batch, bn_gamma, bn_beta,
           W1_rel, W1_root, b1, W2_rel, W2_root, b2,
           W3_rel, W3_root, b3, W_lin, b_lin):
    eps = 1e-5
    gv = (bn_gamma * (1.0 / math.sqrt(1.0 + eps)))[None, :]
    bv = bn_beta[None, :]

    xp = jnp.pad(x, ((0, NP - N), (0, 0)))
    src = jnp.pad(edge_index[0], (0, EP - E)).reshape(NW, NCH, CHUNK)
    dst = jnp.pad(edge_index[1], (0, EP - E), constant_values=N).reshape(NW, NCH, CHUNK)
    segs = jnp.pad(batch, (0, NP - N), constant_values=G).astype(jnp.int32)
    wl = jnp.pad(W_lin, ((0, 0), (0, D - C)))
    bl = jnp.pad(b_lin, (0, D - C))[None, :]

    h = _bn(xp, gv, bv)

    for (wr, wo, b, relu) in (
        (W1_rel, W1_root, b1, True),
        (W2_rel, W2_root, b2, True),
        (W3_rel, W3_root, b3, False),
    ):
        agg = _sc_agg(h, src, dst)
        h = _layer(agg[0], agg[1], h, wr, wo, b[None, :], relu)

    out = _pool(h, segs, wl, bl)
    return out[:, :C]


# speedup vs baseline: 1.5167x; 1.0000x over previous
import functools
import math

import jax
import jax.numpy as jnp
from jax import lax
from jax.experimental import pallas as pl
from jax.experimental.pallas import tpu as pltpu
from jax.experimental.pallas import tpu_sc as plsc

N = 10000
E = 320000
D = 128
G = 64
C = 10

NP = 10240
NW = 32
CHUNK = 128
NCH = 79
EP = NW * NCH * CHUNK
ROWS_PER_SUB = NP // 16
ZR = 16
NZB = ROWS_PER_SUB // ZR


def _sc_agg_body(h_hbm, srcs_hbm, dsts_hbm, out_hbm,
                 src_v, dst_v, rows0_v, rows1_v, zero_v, acc_sh, sem0, sem1, semz):
    c = lax.axis_index("c")
    s = lax.axis_index("s")
    wid = s * 2 + c

    def zrow(i, carry):
        for l in range(8):
            zero_v[i, pl.ds(l * 16, 16)] = jnp.zeros((16,), jnp.float32)
        return carry
    lax.fori_loop(0, ZR, zrow, 0)

    def zblk(j, carry):
        pltpu.sync_copy(zero_v, acc_sh.at[pl.ds(s * ROWS_PER_SUB + j * ZR, ZR)])
        return carry
    lax.fori_loop(0, NZB, zblk, 0)
    plsc.subcore_barrier()

    pltpu.sync_copy(srcs_hbm.at[wid], src_v)
    pltpu.sync_copy(dsts_hbm.at[wid], dst_v)

    def body(j, carry):
        pltpu.async_copy(h_hbm.at[src_v.at[j]], rows0_v, sem0).wait()
        pltpu.sync_copy(rows0_v, acc_sh.at[dst_v.at[j]], add=True)
        return carry
    lax.fori_loop(0, NCH, body, 0)
    plsc.subcore_barrier()

    pltpu.sync_copy(acc_sh.at[pl.ds(s * ROWS_PER_SUB, ROWS_PER_SUB)],
                    out_hbm.at[c].at[pl.ds(s * ROWS_PER_SUB, ROWS_PER_SUB)])


@functools.cache
def _sc_agg_call():
    return pl.kernel(
        _sc_agg_body,
        out_type=jax.ShapeDtypeStruct((2, NP, D), jnp.float32),
        mesh=plsc.VectorSubcoreMesh(core_axis_name="c", subcore_axis_name="s"),
        scratch_types=[
            pltpu.VMEM((NCH, CHUNK), jnp.int32),
            pltpu.VMEM((NCH, CHUNK), jnp.int32),
            pltpu.VMEM((CHUNK, D), jnp.float32),
            pltpu.VMEM((CHUNK, D), jnp.float32),
            pltpu.VMEM((ZR, D), jnp.float32),
            pltpu.VMEM_SHARED((NP, D), jnp.float32),
            pltpu.SemaphoreType.DMA,
            pltpu.SemaphoreType.DMA,
            pltpu.SemaphoreType.DMA,
        ],
    )


def _sc_agg(h, src, dst):
    return _sc_agg_call()(h, src, dst)


BM = 512


def _bn_body(x_ref, g_ref, b_ref, o_ref):
    o_ref[...] = x_ref[...] * g_ref[...] + b_ref[...]


def _bn(x, gv, bv):
    return pl.pallas_call(
        _bn_body,
        grid=(NP // BM,),
        in_specs=[
            pl.BlockSpec((BM, D), lambda i: (i, 0)),
            pl.BlockSpec((1, D), lambda i: (0, 0)),
            pl.BlockSpec((1, D), lambda i: (0, 0)),
        ],
        out_specs=pl.BlockSpec((BM, D), lambda i: (i, 0)),
        out_shape=jax.ShapeDtypeStruct((NP, D), jnp.float32),
    )(x, gv, bv)


def _layer_body(a0_ref, a1_ref, h_ref, wr_ref, wo_ref, b_ref, o_ref, *, relu):
    agg = a0_ref[...] + a1_ref[...]
    acc = jnp.dot(agg, wr_ref[...], preferred_element_type=jnp.float32)
    acc = acc + jnp.dot(h_ref[...], wo_ref[...], preferred_element_type=jnp.float32)
    acc = acc + b_ref[...]
    if relu:
        acc = jnp.maximum(acc, 0.0)
    o_ref[...] = acc


def _layer(a0, a1, h, wr, wo, b, relu):
    return pl.pallas_call(
        functools.partial(_layer_body, relu=relu),
        grid=(NP // BM,),
        in_specs=[
            pl.BlockSpec((BM, D), lambda i: (i, 0)),
            pl.BlockSpec((BM, D), lambda i: (i, 0)),
            pl.BlockSpec((BM, D), lambda i: (i, 0)),
            pl.BlockSpec((D, D), lambda i: (0, 0)),
            pl.BlockSpec((D, D), lambda i: (0, 0)),
            pl.BlockSpec((1, D), lambda i: (0, 0)),
        ],
        out_specs=pl.BlockSpec((BM, D), lambda i: (i, 0)),
        out_shape=jax.ShapeDtypeStruct((NP, D), jnp.float32),
    )(a0, a1, h, wr, wo, b)


def _pool_body(h_ref, b_ref, wl_ref, bl_ref, o_ref, sums_ref, cnts_ref):
    i = pl.program_id(0)

    @pl.when(i == 0)
    def _():
        sums_ref[...] = jnp.zeros_like(sums_ref)
        cnts_ref[...] = jnp.zeros_like(cnts_ref)

    seg = b_ref[...]
    onehot = jnp.where(
        seg[:, None] == lax.broadcasted_iota(jnp.int32, (1, G), 1),
        1.0, 0.0).astype(jnp.float32)
    sums_ref[...] += lax.dot_general(
        onehot, h_ref[...], (((0,), (0,)), ((), ())),
        preferred_element_type=jnp.float32)
    cnts_ref[...] += jnp.sum(onehot, axis=0)[:, None]

    @pl.when(i == pl.num_programs(0) - 1)
    def _():
        pooled = sums_ref[...] / jnp.maximum(cnts_ref[...], 1.0)
        o_ref[...] = jnp.dot(pooled, wl_ref[...],
                             preferred_element_type=jnp.float32) + bl_ref[...]


def _pool(h, segs, wl, bl):
    return pl.pallas_call(
        _pool_body,
        grid=(NP // BM,),
        in_specs=[
            pl.BlockSpec((BM, D), lambda i: (i, 0)),
            pl.BlockSpec((BM,), lambda i: (i,)),
            pl.BlockSpec((D, D), lambda i: (0, 0)),
            pl.BlockSpec((1, D), lambda i: (0, 0)),
        ],
        out_specs=pl.BlockSpec((G, D), lambda i: (0, 0)),
        out_shape=jax.ShapeDtypeStruct((G, D), jnp.float32),
        scratch_shapes=[
            pltpu.VMEM((G, D), jnp.float32),
            pltpu.VMEM((G, D), jnp.float32),
        ],
    )(h, segs, wl, bl)


def kernel(x, edge_index, batch, bn_gamma, bn_beta,
           W1_rel, W1_root, b1, W2_rel, W2_root, b2,
           W3_rel, W3_root, b3, W_lin, b_lin):
    eps = 1e-5
    gv = (bn_gamma * (1.0 / math.sqrt(1.0 + eps)))[None, :]
    bv = bn_beta[None, :]

    xp = jnp.pad(x, ((0, NP - N), (0, 0)))
    src = jnp.pad(edge_index[0], (0, EP - E)).reshape(NW, NCH, CHUNK)
    dst = jnp.pad(edge_index[1], (0, EP - E), constant_values=N).reshape(NW, NCH, CHUNK)
    segs = jnp.pad(batch, (0, NP - N), constant_values=G).astype(jnp.int32)
    wl = jnp.pad(W_lin, ((0, 0), (0, D - C)))
    bl = jnp.pad(b_lin, (0, D - C))[None, :]

    h = _bn(xp, gv, bv)

    for (wr, wo, b, relu) in (
        (W1_rel, W1_root, b1, True),
        (W2_rel, W2_root, b2, True),
        (W3_rel, W3_root, b3, False),
    ):
        agg = _sc_agg(h, src, dst)
        h = _layer(agg[0], agg[1], h, wr, wo, b[None, :], relu)

    out = _pool(h, segs, wl, bl)
    return out[:, :C]
